# Initial kernel scaffold; baseline (speedup 1.0000x reference)
import jax
import jax.numpy as jnp
from jax.experimental import pallas as pl


def kernel(points1, points2, features1, features2, W0, b0, gamma0, beta0, W1, b1, gamma1, beta1, W2, b2, gamma2, beta2):
    raise NotImplementedError("write your pallas kernel here")


# speedup vs baseline: 10.7470x; 10.7470x over previous
import functools

import jax
import jax.numpy as jnp
from jax import lax
from jax.experimental import pallas as pl
from jax.experimental.pallas import tpu as pltpu
from jax.experimental.pallas import tpu_sc as plsc

_EPS = 1e-3
_K = 16
_NC, _NS = 2, 16
_NW = _NC * _NS
_CHUNK = 128


def _tables_kernel(x2_ref, p2_ref, x1_ref, p1_ref, wf2_ref, wrel_ref,
                   wf1_ref, b0_ref, t2_ref, f1b_ref):
    t2_ref[...] = (
        jnp.dot(x2_ref[...], wf2_ref[...], preferred_element_type=jnp.float32)
        + jnp.dot(p2_ref[...], wrel_ref[...], preferred_element_type=jnp.float32))
    f1b_ref[...] = (
        jnp.dot(x1_ref[...], wf1_ref[...], preferred_element_type=jnp.float32)
        - jnp.dot(p1_ref[...], wrel_ref[...], preferred_element_type=jnp.float32)
        + b0_ref[...])


def _knn_kernel(p1_ref, p2t_ref, idx_ref, *, n, k, t):
    q = p1_ref[0]
    pt = p2t_ref[0]
    qn = jnp.sum(q * q, axis=1, keepdims=True)
    pn = jnp.sum(pt * pt, axis=0, keepdims=True)
    dist = qn - 2.0 * jnp.dot(q, pt, preferred_element_type=jnp.float32) + pn
    iota = lax.broadcasted_iota(jnp.int32, (t, n), 1)
    base = pl.program_id(0) * n
    cols = []
    for _ in range(k):
        m = jnp.min(dist, axis=1, keepdims=True)
        cand = jnp.where(dist == m, iota, n)
        sel = jnp.min(cand, axis=1, keepdims=True)
        cols.append(sel)
        dist = jnp.where(iota == sel, jnp.float32(jnp.inf), dist)
    idx_ref[0] = jnp.concatenate(cols, axis=1) + base


def _sc_gather(idx3, table):
    nw, nch, chunk = idx3.shape
    c = table.shape[1]
    bpw = nch * chunk
    mesh = plsc.VectorSubcoreMesh(core_axis_name="c", subcore_axis_name="s",
                                  num_cores=_NC, num_subcores=_NS)

    def body(idx_hbm, table_hbm, out_hbm, i0, i1, r0, r1, sg0, sg1, ss0, ss1):
        wid = lax.axis_index("s") * _NC + lax.axis_index("c")
        base = wid * bpw

        def do_pair(p, drain_prev):
            c0 = 2 * p
            pltpu.sync_copy(idx_hbm.at[wid, c0], i0)
            g0 = pltpu.async_copy(table_hbm.at[i0], r0, sg0)
            if drain_prev:
                pltpu.make_async_copy(
                    r1, out_hbm.at[pl.ds(base, chunk)], ss1).wait()
            g0.wait()
            s0 = pltpu.async_copy(
                r0, out_hbm.at[pl.ds(base + c0 * chunk, chunk)], ss0)
            pltpu.sync_copy(idx_hbm.at[wid, c0 + 1], i1)
            g1 = pltpu.async_copy(table_hbm.at[i1], r1, sg1)
            g1.wait()
            s0.wait()
            pltpu.async_copy(
                r1, out_hbm.at[pl.ds(base + (c0 + 1) * chunk, chunk)], ss1)

        do_pair(0, False)

        @pl.loop(1, nch // 2)
        def _pairs(p):
            do_pair(p, True)

        pltpu.make_async_copy(r1, out_hbm.at[pl.ds(base, chunk)], ss1).wait()

    fn = pl.kernel(
        body,
        out_type=jax.ShapeDtypeStruct((nw * bpw, c), jnp.float32),
        mesh=mesh,
        scratch_types=[
            pltpu.VMEM((chunk,), jnp.int32),
            pltpu.VMEM((chunk,), jnp.int32),
            pltpu.VMEM((chunk, c), jnp.float32),
            pltpu.VMEM((chunk, c), jnp.float32),
            pltpu.SemaphoreType.DMA,
            pltpu.SemaphoreType.DMA,
            pltpu.SemaphoreType.DMA,
            pltpu.SemaphoreType.DMA,
        ],
    )
    return fn(idx3, table)


def _stats0_kernel(t2g_ref, f1b_ref, s_ref, ss_ref):
    y = t2g_ref[...] + f1b_ref[...][:, None, :]
    s = jnp.sum(jnp.sum(y, axis=1), axis=0, keepdims=True)
    ss = jnp.sum(jnp.sum(y * y, axis=1), axis=0, keepdims=True)

    @pl.when(pl.program_id(0) == 0)
    def _():
        s_ref[...] = jnp.zeros_like(s_ref)
        ss_ref[...] = jnp.zeros_like(ss_ref)

    s_ref[...] += s
    ss_ref[...] += ss


def _layer1_kernel(t2g_ref, f1b_ref, sc_ref, sh_ref, w_ref, b_ref,
                   y_ref, s_ref, ss_ref, *, gt, k, c):
    y0 = t2g_ref[...] + f1b_ref[...][:, None, :]
    h = jnp.maximum(y0 * sc_ref[...][None] + sh_ref[...][None], 0.0)
    y = jnp.dot(h.reshape(gt * k, c), w_ref[...],
                preferred_element_type=jnp.float32) + b_ref[...]
    y_ref[...] = y.reshape(gt, k, c)
    s = jnp.sum(y, axis=0, keepdims=True)
    ss = jnp.sum(y * y, axis=0, keepdims=True)

    @pl.when(pl.program_id(0) == 0)
    def _():
        s_ref[...] = jnp.zeros_like(s_ref)
        ss_ref[...] = jnp.zeros_like(ss_ref)

    s_ref[...] += s
    ss_ref[...] += ss


def _layer2_kernel(yin_ref, sc_ref, sh_ref, w_ref, b_ref,
                   y_ref, s_ref, ss_ref, *, gt, k, c):
    h = jnp.maximum(yin_ref[...] * sc_ref[...][None] + sh_ref[...][None], 0.0)
    y = jnp.dot(h.reshape(gt * k, c), w_ref[...],
                preferred_element_type=jnp.float32) + b_ref[...]
    y_ref[...] = y.reshape(gt, k, c)
    s = jnp.sum(y, axis=0, keepdims=True)
    ss = jnp.sum(y * y, axis=0, keepdims=True)

    @pl.when(pl.program_id(0) == 0)
    def _():
        s_ref[...] = jnp.zeros_like(s_ref)
        ss_ref[...] = jnp.zeros_like(ss_ref)

    s_ref[...] += s
    ss_ref[...] += ss


def _final_kernel(yin_ref, sc_ref, sh_ref, out_ref):
    h = jnp.maximum(yin_ref[...] * sc_ref[...][None] + sh_ref[...][None], 0.0)
    out_ref[...] = jnp.max(h, axis=1)


def kernel(points1, points2, features1, features2,
           W0, b0, gamma0, beta0,
           W1, b1, gamma1, beta1,
           W2, b2, gamma2, beta2):
    B, N, _ = points1.shape
    C = features1.shape[1]
    K = _K
    BN = B * N
    R = BN * K
    f32 = jnp.float32

    x2 = jnp.transpose(features2, (0, 2, 1)).reshape(BN, C)
    x1 = jnp.transpose(features1, (0, 2, 1)).reshape(BN, C)
    p2r = points2.reshape(BN, 3)
    p1r = points1.reshape(BN, 3)
    wrel = jnp.transpose(W0[:, :3])
    wf2 = jnp.transpose(W0[:, 3:3 + C])
    wf1 = jnp.transpose(W0[:, 3 + C:])
    b0r = b0.reshape(1, C)

    RT = 2048
    t2, f1b = pl.pallas_call(
        _tables_kernel,
        grid=(BN // RT,),
        in_specs=[
            pl.BlockSpec((RT, C), lambda i: (i, 0)),
            pl.BlockSpec((RT, 3), lambda i: (i, 0)),
            pl.BlockSpec((RT, C), lambda i: (i, 0)),
            pl.BlockSpec((RT, 3), lambda i: (i, 0)),
            pl.BlockSpec((C, C), lambda i: (0, 0)),
            pl.BlockSpec((3, C), lambda i: (0, 0)),
            pl.BlockSpec((C, C), lambda i: (0, 0)),
            pl.BlockSpec((1, C), lambda i: (0, 0)),
        ],
        out_specs=[pl.BlockSpec((RT, C), lambda i: (i, 0)),
                   pl.BlockSpec((RT, C), lambda i: (i, 0))],
        out_shape=[jax.ShapeDtypeStruct((BN, C), f32),
                   jax.ShapeDtypeStruct((BN, C), f32)],
    )(x2, p2r, x1, p1r, wf2, wrel, wf1, b0r)

    T = 256
    p2t = jnp.transpose(points2, (0, 2, 1))
    idx = pl.pallas_call(
        functools.partial(_knn_kernel, n=N, k=K, t=T),
        grid=(B, N // T),
        in_specs=[
            pl.BlockSpec((1, T, 3), lambda b, i: (b, i, 0)),
            pl.BlockSpec((1, 3, N), lambda b, i: (b, 0, 0)),
        ],
        out_specs=pl.BlockSpec((1, T, K), lambda b, i: (b, i, 0)),
        out_shape=jax.ShapeDtypeStruct((B, N, K), jnp.int32),
    )(points1, p2t)

    idx3 = idx.reshape(_NW, R // (_NW * _CHUNK), _CHUNK)
    t2g3 = _sc_gather(idx3, t2).reshape(BN, K, C)

    GT = 128
    grid = (BN // GT,)
    in3 = pl.BlockSpec((GT, K, C), lambda i: (i, 0, 0))
    in2 = pl.BlockSpec((GT, C), lambda i: (i, 0))
    vec = pl.BlockSpec((1, C), lambda i: (0, 0))
    vec_shape = jax.ShapeDtypeStruct((1, C), f32)

    s0, ss0 = pl.pallas_call(
        _stats0_kernel,
        grid=grid,
        in_specs=[in3, in2],
        out_specs=[vec, vec],
        out_shape=[vec_shape, vec_shape],
    )(t2g3, f1b)

    def _affine(s, ss, gamma, beta):
        mean = s / R
        var = ss / R - mean * mean
        scale = gamma.reshape(1, C) / jnp.sqrt(var + _EPS)
        shift = beta.reshape(1, C) - mean * scale
        return scale, shift

    sc0, sh0 = _affine(s0, ss0, gamma0, beta0)

    y1, s1, ss1 = pl.pallas_call(
        functools.partial(_layer1_kernel, gt=GT, k=K, c=C),
        grid=grid,
        in_specs=[in3, in2, vec, vec,
                  pl.BlockSpec((C, C), lambda i: (0, 0)), vec],
        out_specs=[in3, vec, vec],
        out_shape=[jax.ShapeDtypeStruct((BN, K, C), f32), vec_shape, vec_shape],
    )(t2g3, f1b, sc0, sh0, jnp.transpose(W1), b1.reshape(1, C))

    sc1, sh1 = _affine(s1, ss1, gamma1, beta1)

    y2, s2, ss2 = pl.pallas_call(
        functools.partial(_layer2_kernel, gt=GT, k=K, c=C),
        grid=grid,
        in_specs=[in3, vec, vec,
                  pl.BlockSpec((C, C), lambda i: (0, 0)), vec],
        out_specs=[in3, vec, vec],
        out_shape=[jax.ShapeDtypeStruct((BN, K, C), f32), vec_shape, vec_shape],
    )(y1, sc1, sh1, jnp.transpose(W2), b2.reshape(1, C))

    sc2, sh2 = _affine(s2, ss2, gamma2, beta2)

    outr = pl.pallas_call(
        _final_kernel,
        grid=grid,
        in_specs=[in3, vec, vec],
        out_specs=in2,
        out_shape=jax.ShapeDtypeStruct((BN, C), f32),
    )(y2, sc2, sh2)

    return jnp.transpose(outr.reshape(B, N, C), (0, 2, 1))


# speedup vs baseline: 12.5769x; 1.1703x over previous
import functools

import jax
import jax.numpy as jnp
from jax import lax
from jax.experimental import pallas as pl
from jax.experimental.pallas import tpu as pltpu
from jax.experimental.pallas import tpu_sc as plsc

_EPS = 1e-3
_K = 16
_NC, _NS = 2, 16
_NW = _NC * _NS
_CHUNK = 128


def _tables_kernel(x2_ref, p2_ref, x1_ref, p1_ref, wf2_ref, wrel_ref,
                   wf1_ref, b0_ref, t2_ref, f1b_ref):
    t2_ref[...] = (
        jnp.dot(x2_ref[...], wf2_ref[...], preferred_element_type=jnp.float32)
        + jnp.dot(p2_ref[...], wrel_ref[...], preferred_element_type=jnp.float32))
    f1b_ref[...] = (
        jnp.dot(x1_ref[...], wf1_ref[...], preferred_element_type=jnp.float32)
        - jnp.dot(p1_ref[...], wrel_ref[...], preferred_element_type=jnp.float32)
        + b0_ref[...])


def _knn_kernel(p1_ref, p2t_ref, idx_ref, *, n, k, t):
    q = p1_ref[0]
    pt = p2t_ref[0]
    qn = jnp.sum(q * q, axis=1, keepdims=True)
    pn = jnp.sum(pt * pt, axis=0, keepdims=True)
    dist = qn - 2.0 * jnp.dot(q, pt, preferred_element_type=jnp.float32) + pn
    iota = lax.broadcasted_iota(jnp.int32, (t, n), 1)
    big = jnp.int32(n)
    base = pl.program_id(0) * n
    cols = []
    for _ in range(k):
        m = jnp.min(dist, axis=1, keepdims=True)
        cand = jnp.where(dist == m, iota, big)
        sel = jnp.min(cand, axis=1, keepdims=True)
        cols.append(sel)
        dist = jnp.where(iota == sel, jnp.float32(jnp.inf), dist)
    idx_ref[0] = jnp.concatenate(cols, axis=1).astype(jnp.int32) + base


def _sc_gather(idx3, table):
    nw, nch, chunk = idx3.shape
    c = table.shape[1]
    bpw = nch * chunk
    mesh = plsc.VectorSubcoreMesh(core_axis_name="c", subcore_axis_name="s",
                                  num_cores=_NC, num_subcores=_NS)

    def body(idx_hbm, table_hbm, out_hbm, i0, i1, r0, r1, sg0, sg1, ss0, ss1):
        wid = lax.axis_index("s") * _NC + lax.axis_index("c")
        base = wid * bpw

        def do_pair(p, drain_prev):
            c0 = 2 * p
            pltpu.sync_copy(idx_hbm.at[wid, c0], i0)
            g0 = pltpu.async_copy(table_hbm.at[i0], r0, sg0)
            if drain_prev:
                pltpu.make_async_copy(
                    r1, out_hbm.at[pl.ds(base, chunk)], ss1).wait()
            g0.wait()
            s0 = pltpu.async_copy(
                r0, out_hbm.at[pl.ds(base + c0 * chunk, chunk)], ss0)
            pltpu.sync_copy(idx_hbm.at[wid, c0 + 1], i1)
            g1 = pltpu.async_copy(table_hbm.at[i1], r1, sg1)
            g1.wait()
            s0.wait()
            pltpu.async_copy(
                r1, out_hbm.at[pl.ds(base + (c0 + 1) * chunk, chunk)], ss1)

        do_pair(0, False)

        @pl.loop(1, nch // 2)
        def _pairs(p):
            do_pair(p, True)

        pltpu.make_async_copy(r1, out_hbm.at[pl.ds(base, chunk)], ss1).wait()

    fn = pl.kernel(
        body,
        out_type=jax.ShapeDtypeStruct((nw * bpw, c), jnp.float32),
        mesh=mesh,
        scratch_types=[
            pltpu.VMEM((chunk,), jnp.int32),
            pltpu.VMEM((chunk,), jnp.int32),
            pltpu.VMEM((chunk, c), jnp.float32),
            pltpu.VMEM((chunk, c), jnp.float32),
            pltpu.SemaphoreType.DMA,
            pltpu.SemaphoreType.DMA,
            pltpu.SemaphoreType.DMA,
            pltpu.SemaphoreType.DMA,
        ],
    )
    return fn(idx3, table)


def _stats0_kernel(t2g_ref, f1b_ref, s_ref, ss_ref):
    y = t2g_ref[...] + f1b_ref[...][:, None, :]
    s = jnp.sum(jnp.sum(y, axis=1), axis=0, keepdims=True)
    ss = jnp.sum(jnp.sum(y * y, axis=1), axis=0, keepdims=True)

    @pl.when(pl.program_id(0) == 0)
    def _():
        s_ref[...] = jnp.zeros_like(s_ref)
        ss_ref[...] = jnp.zeros_like(ss_ref)

    s_ref[...] += s
    ss_ref[...] += ss


def _layer1_kernel(t2g_ref, f1b_ref, sc_ref, sh_ref, w_ref, b_ref,
                   y_ref, s_ref, ss_ref, *, gt, k, c):
    y0 = t2g_ref[...] + f1b_ref[...][:, None, :]
    h = jnp.maximum(y0 * sc_ref[...][None] + sh_ref[...][None], 0.0)
    y = jnp.dot(h.reshape(gt * k, c).astype(jnp.bfloat16), w_ref[...],
                preferred_element_type=jnp.float32) + b_ref[...]
    y_ref[...] = y.reshape(gt, k, c).astype(jnp.bfloat16)
    s = jnp.sum(y, axis=0, keepdims=True)
    ss = jnp.sum(y * y, axis=0, keepdims=True)

    @pl.when(pl.program_id(0) == 0)
    def _():
        s_ref[...] = jnp.zeros_like(s_ref)
        ss_ref[...] = jnp.zeros_like(ss_ref)

    s_ref[...] += s
    ss_ref[...] += ss


def _layer2_kernel(yin_ref, sc_ref, sh_ref, w_ref, b_ref,
                   y_ref, s_ref, ss_ref, *, gt, k, c):
    yin = yin_ref[...].astype(jnp.float32)
    h = jnp.maximum(yin * sc_ref[...][None] + sh_ref[...][None], 0.0)
    y = jnp.dot(h.reshape(gt * k, c).astype(jnp.bfloat16), w_ref[...],
                preferred_element_type=jnp.float32) + b_ref[...]
    y_ref[...] = y.reshape(gt, k, c).astype(jnp.bfloat16)
    s = jnp.sum(y, axis=0, keepdims=True)
    ss = jnp.sum(y * y, axis=0, keepdims=True)

    @pl.when(pl.program_id(0) == 0)
    def _():
        s_ref[...] = jnp.zeros_like(s_ref)
        ss_ref[...] = jnp.zeros_like(ss_ref)

    s_ref[...] += s
    ss_ref[...] += ss


def _final_kernel(yin_ref, sc_ref, sh_ref, out_ref):
    yin = yin_ref[...].astype(jnp.float32)
    h = jnp.maximum(yin * sc_ref[...][None] + sh_ref[...][None], 0.0)
    out_ref[...] = jnp.max(h, axis=1)


def kernel(points1, points2, features1, features2,
           W0, b0, gamma0, beta0,
           W1, b1, gamma1, beta1,
           W2, b2, gamma2, beta2):
    B, N, _ = points1.shape
    C = features1.shape[1]
    K = _K
    BN = B * N
    R = BN * K
    f32 = jnp.float32

    x2 = jnp.transpose(features2, (0, 2, 1)).reshape(BN, C)
    x1 = jnp.transpose(features1, (0, 2, 1)).reshape(BN, C)
    p2r = points2.reshape(BN, 3)
    p1r = points1.reshape(BN, 3)
    wrel = jnp.transpose(W0[:, :3])
    wf2 = jnp.transpose(W0[:, 3:3 + C])
    wf1 = jnp.transpose(W0[:, 3 + C:])
    b0r = b0.reshape(1, C)

    RT = 2048
    t2, f1b = pl.pallas_call(
        _tables_kernel,
        grid=(BN // RT,),
        in_specs=[
            pl.BlockSpec((RT, C), lambda i: (i, 0)),
            pl.BlockSpec((RT, 3), lambda i: (i, 0)),
            pl.BlockSpec((RT, C), lambda i: (i, 0)),
            pl.BlockSpec((RT, 3), lambda i: (i, 0)),
            pl.BlockSpec((C, C), lambda i: (0, 0)),
            pl.BlockSpec((3, C), lambda i: (0, 0)),
            pl.BlockSpec((C, C), lambda i: (0, 0)),
            pl.BlockSpec((1, C), lambda i: (0, 0)),
        ],
        out_specs=[pl.BlockSpec((RT, C), lambda i: (i, 0)),
                   pl.BlockSpec((RT, C), lambda i: (i, 0))],
        out_shape=[jax.ShapeDtypeStruct((BN, C), f32),
                   jax.ShapeDtypeStruct((BN, C), f32)],
    )(x2, p2r, x1, p1r, wf2, wrel, wf1, b0r)

    T = 256
    p2t = jnp.transpose(points2, (0, 2, 1))
    idx = pl.pallas_call(
        functools.partial(_knn_kernel, n=N, k=K, t=T),
        grid=(B, N // T),
        in_specs=[
            pl.BlockSpec((1, T, 3), lambda b, i: (b, i, 0)),
            pl.BlockSpec((1, 3, N), lambda b, i: (b, 0, 0)),
        ],
        out_specs=pl.BlockSpec((1, T, K), lambda b, i: (b, i, 0)),
        out_shape=jax.ShapeDtypeStruct((B, N, K), jnp.int32),
    )(points1, p2t)

    idx3 = idx.reshape(_NW, R // (_NW * _CHUNK), _CHUNK)
    t2g3 = _sc_gather(idx3, t2).reshape(BN, K, C)

    GT = 256
    grid = (BN // GT,)
    in3 = pl.BlockSpec((GT, K, C), lambda i: (i, 0, 0))
    in2 = pl.BlockSpec((GT, C), lambda i: (i, 0))
    vec = pl.BlockSpec((1, C), lambda i: (0, 0))
    vec_shape = jax.ShapeDtypeStruct((1, C), f32)

    s0, ss0 = pl.pallas_call(
        _stats0_kernel,
        grid=grid,
        in_specs=[in3, in2],
        out_specs=[vec, vec],
        out_shape=[vec_shape, vec_shape],
    )(t2g3, f1b)

    def _affine(s, ss, gamma, beta):
        mean = s / R
        var = ss / R - mean * mean
        scale = gamma.reshape(1, C) / jnp.sqrt(var + _EPS)
        shift = beta.reshape(1, C) - mean * scale
        return scale, shift

    sc0, sh0 = _affine(s0, ss0, gamma0, beta0)

    y1, s1, ss1 = pl.pallas_call(
        functools.partial(_layer1_kernel, gt=GT, k=K, c=C),
        grid=grid,
        in_specs=[in3, in2, vec, vec,
                  pl.BlockSpec((C, C), lambda i: (0, 0)), vec],
        out_specs=[in3, vec, vec],
        out_shape=[jax.ShapeDtypeStruct((BN, K, C), jnp.bfloat16),
                   vec_shape, vec_shape],
    )(t2g3, f1b, sc0, sh0, jnp.transpose(W1).astype(jnp.bfloat16),
      b1.reshape(1, C))

    sc1, sh1 = _affine(s1, ss1, gamma1, beta1)

    y2, s2, ss2 = pl.pallas_call(
        functools.partial(_layer2_kernel, gt=GT, k=K, c=C),
        grid=grid,
        in_specs=[in3, vec, vec,
                  pl.BlockSpec((C, C), lambda i: (0, 0)), vec],
        out_specs=[in3, vec, vec],
        out_shape=[jax.ShapeDtypeStruct((BN, K, C), jnp.bfloat16),
                   vec_shape, vec_shape],
    )(y1, sc1, sh1, jnp.transpose(W2).astype(jnp.bfloat16),
      b2.reshape(1, C))

    sc2, sh2 = _affine(s2, ss2, gamma2, beta2)

    outr = pl.pallas_call(
        _final_kernel,
        grid=grid,
        in_specs=[in3, vec, vec],
        out_specs=in2,
        out_shape=jax.ShapeDtypeStruct((BN, C), f32),
    )(y2, sc2, sh2)

    return jnp.transpose(outr.reshape(B, N, C), (0, 2, 1))


# speedup vs baseline: 15.4375x; 1.2274x over previous
import functools

import jax
import jax.numpy as jnp
from jax import lax
from jax.experimental import pallas as pl
from jax.experimental.pallas import tpu as pltpu
from jax.experimental.pallas import tpu_sc as plsc

_EPS = 1e-3
_K = 16
_NC, _NS = 2, 16
_NW = _NC * _NS
_CHUNK = 128


def _tables_kernel(x2_ref, p2_ref, x1_ref, p1_ref, wf2_ref, wrel_ref,
                   wf1_ref, b0_ref, t2_ref, f1b_ref):
    t2_ref[...] = (
        jnp.dot(x2_ref[...], wf2_ref[...], preferred_element_type=jnp.float32)
        + jnp.dot(p2_ref[...], wrel_ref[...], preferred_element_type=jnp.float32))
    f1b_ref[...] = (
        jnp.dot(x1_ref[...], wf1_ref[...], preferred_element_type=jnp.float32)
        - jnp.dot(p1_ref[...], wrel_ref[...], preferred_element_type=jnp.float32)
        + b0_ref[...])


def _oddeven_merge(lo, hi, r):
    step = r * 2
    if step < hi - lo:
        yield from _oddeven_merge(lo, hi, step)
        yield from _oddeven_merge(lo + r, hi, step)
        yield from ((i, i + r) for i in range(lo + r, hi - r, step))
    else:
        yield (lo, lo + r)


def _oddeven_sort(lo, hi):
    if (hi - lo) >= 1:
        mid = lo + ((hi - lo) // 2)
        yield from _oddeven_sort(lo, mid)
        yield from _oddeven_sort(mid + 1, hi)
        yield from _oddeven_merge(lo, hi, 1)


_BATCHER16 = list(_oddeven_sort(0, 15))
_DEPTH = 6


def _knn_kernel(p1_ref, p2t_ref, idx_ref, *, n, k, t):
    q = p1_ref[0]
    pt = p2t_ref[0]
    qn = jnp.sum(q * q, axis=1, keepdims=True)
    pn = jnp.sum(pt * pt, axis=0, keepdims=True)
    dist = qn - 2.0 * jnp.dot(q, pt, preferred_element_type=jnp.float32) + pn
    g = 16
    l = n // g
    lane = lax.broadcasted_iota(jnp.int32, (t, l), 1)
    inf = jnp.float32(jnp.inf)
    big = jnp.int32(n)
    base = pl.program_id(0) * n

    svals = [dist[:, j * l:(j + 1) * l] for j in range(g)]
    sidx = [lane + (j * l) for j in range(g)]
    for a, b in _BATCHER16:
        swap = svals[a] > svals[b]
        va, vb = svals[a], svals[b]
        svals[a] = jnp.where(swap, vb, va)
        svals[b] = jnp.where(swap, va, vb)
        ia, ib = sidx[a], sidx[b]
        sidx[a] = jnp.where(swap, ib, ia)
        sidx[b] = jnp.where(swap, ia, ib)

    heads = svals[0]
    hflat = sidx[0]
    pcol = jnp.zeros((t, l), jnp.int32)
    outs = []
    for _ in range(k):
        m = jnp.min(heads, axis=1, keepdims=True)
        csel = jnp.where(heads == m, hflat, big)
        selflat = jnp.min(csel, axis=1, keepdims=True)
        outs.append(selflat)
        hit = csel == selflat
        pcol = pcol + hit.astype(jnp.int32)
        nh = jnp.full((t, l), inf)
        nf = jnp.full((t, l), big)
        for j in range(_DEPTH - 1, 0, -1):
            pj = pcol == j
            nh = jnp.where(pj, svals[j], nh)
            nf = jnp.where(pj, sidx[j], nf)
        heads = jnp.where(hit, nh, heads)
        hflat = jnp.where(hit, nf, hflat)

    idx_ref[0] = jnp.concatenate(outs, axis=1) + base

    @pl.when(jnp.any(pcol >= _DEPTH))
    def _():
        d = dist
        iota = lax.broadcasted_iota(jnp.int32, (t, n), 1)
        cols = []
        for _ in range(k):
            mm = jnp.min(d, axis=1, keepdims=True)
            cand = jnp.where(d == mm, iota, big)
            sel = jnp.min(cand, axis=1, keepdims=True)
            cols.append(sel)
            d = jnp.where(iota == sel, inf, d)
        idx_ref[0] = jnp.concatenate(cols, axis=1) + base


def _sc_gather(idx3, table):
    nw, nch, chunk = idx3.shape
    c = table.shape[1]
    bpw = nch * chunk
    mesh = plsc.VectorSubcoreMesh(core_axis_name="c", subcore_axis_name="s",
                                  num_cores=_NC, num_subcores=_NS)

    def body(idx_hbm, table_hbm, out_hbm, i0, i1, r0, r1, sg0, sg1, ss0, ss1):
        wid = lax.axis_index("s") * _NC + lax.axis_index("c")
        base = wid * bpw

        def do_pair(p, drain_prev):
            c0 = 2 * p
            pltpu.sync_copy(idx_hbm.at[wid, c0], i0)
            g0 = pltpu.async_copy(table_hbm.at[i0], r0, sg0)
            if drain_prev:
                pltpu.make_async_copy(
                    r1, out_hbm.at[pl.ds(base, chunk)], ss1).wait()
            g0.wait()
            s0 = pltpu.async_copy(
                r0, out_hbm.at[pl.ds(base + c0 * chunk, chunk)], ss0)
            pltpu.sync_copy(idx_hbm.at[wid, c0 + 1], i1)
            g1 = pltpu.async_copy(table_hbm.at[i1], r1, sg1)
            g1.wait()
            s0.wait()
            pltpu.async_copy(
                r1, out_hbm.at[pl.ds(base + (c0 + 1) * chunk, chunk)], ss1)

        do_pair(0, False)

        @pl.loop(1, nch // 2)
        def _pairs(p):
            do_pair(p, True)

        pltpu.make_async_copy(r1, out_hbm.at[pl.ds(base, chunk)], ss1).wait()

    fn = pl.kernel(
        body,
        out_type=jax.ShapeDtypeStruct((nw * bpw, c), jnp.float32),
        mesh=mesh,
        scratch_types=[
            pltpu.VMEM((chunk,), jnp.int32),
            pltpu.VMEM((chunk,), jnp.int32),
            pltpu.VMEM((chunk, c), jnp.float32),
            pltpu.VMEM((chunk, c), jnp.float32),
            pltpu.SemaphoreType.DMA,
            pltpu.SemaphoreType.DMA,
            pltpu.SemaphoreType.DMA,
            pltpu.SemaphoreType.DMA,
        ],
    )
    return fn(idx3, table)


def _stats0_kernel(t2g_ref, f1b_ref, s_ref, ss_ref):
    y = t2g_ref[...] + f1b_ref[...][:, None, :]
    s = jnp.sum(jnp.sum(y, axis=1), axis=0, keepdims=True)
    ss = jnp.sum(jnp.sum(y * y, axis=1), axis=0, keepdims=True)

    @pl.when(pl.program_id(0) == 0)
    def _():
        s_ref[...] = jnp.zeros_like(s_ref)
        ss_ref[...] = jnp.zeros_like(ss_ref)

    s_ref[...] += s
    ss_ref[...] += ss


def _layer1_kernel(t2g_ref, f1b_ref, sc_ref, sh_ref, w_ref, b_ref,
                   y_ref, s_ref, ss_ref, *, gt, k, c):
    y0 = t2g_ref[...] + f1b_ref[...][:, None, :]
    h = jnp.maximum(y0 * sc_ref[...][None] + sh_ref[...][None], 0.0)
    y = jnp.dot(h.reshape(gt * k, c).astype(jnp.bfloat16), w_ref[...],
                preferred_element_type=jnp.float32) + b_ref[...]
    y_ref[...] = y.reshape(gt, k, c).astype(jnp.bfloat16)
    s = jnp.sum(y, axis=0, keepdims=True)
    ss = jnp.sum(y * y, axis=0, keepdims=True)

    @pl.when(pl.program_id(0) == 0)
    def _():
        s_ref[...] = jnp.zeros_like(s_ref)
        ss_ref[...] = jnp.zeros_like(ss_ref)

    s_ref[...] += s
    ss_ref[...] += ss


def _layer2_kernel(yin_ref, sc_ref, sh_ref, w_ref, b_ref,
                   y_ref, s_ref, ss_ref, *, gt, k, c):
    yin = yin_ref[...].astype(jnp.float32)
    h = jnp.maximum(yin * sc_ref[...][None] + sh_ref[...][None], 0.0)
    y = jnp.dot(h.reshape(gt * k, c).astype(jnp.bfloat16), w_ref[...],
                preferred_element_type=jnp.float32) + b_ref[...]
    y_ref[...] = y.reshape(gt, k, c).astype(jnp.bfloat16)
    s = jnp.sum(y, axis=0, keepdims=True)
    ss = jnp.sum(y * y, axis=0, keepdims=True)

    @pl.when(pl.program_id(0) == 0)
    def _():
        s_ref[...] = jnp.zeros_like(s_ref)
        ss_ref[...] = jnp.zeros_like(ss_ref)

    s_ref[...] += s
    ss_ref[...] += ss


def _final_kernel(yin_ref, sc_ref, sh_ref, out_ref):
    yin = yin_ref[...].astype(jnp.float32)
    h = jnp.maximum(yin * sc_ref[...][None] + sh_ref[...][None], 0.0)
    out_ref[...] = jnp.max(h, axis=1)


def kernel(points1, points2, features1, features2,
           W0, b0, gamma0, beta0,
           W1, b1, gamma1, beta1,
           W2, b2, gamma2, beta2):
    B, N, _ = points1.shape
    C = features1.shape[1]
    K = _K
    BN = B * N
    R = BN * K
    f32 = jnp.float32

    x2 = jnp.transpose(features2, (0, 2, 1)).reshape(BN, C)
    x1 = jnp.transpose(features1, (0, 2, 1)).reshape(BN, C)
    p2r = points2.reshape(BN, 3)
    p1r = points1.reshape(BN, 3)
    wrel = jnp.transpose(W0[:, :3])
    wf2 = jnp.transpose(W0[:, 3:3 + C])
    wf1 = jnp.transpose(W0[:, 3 + C:])
    b0r = b0.reshape(1, C)

    RT = 2048
    t2, f1b = pl.pallas_call(
        _tables_kernel,
        grid=(BN // RT,),
        in_specs=[
            pl.BlockSpec((RT, C), lambda i: (i, 0)),
            pl.BlockSpec((RT, 3), lambda i: (i, 0)),
            pl.BlockSpec((RT, C), lambda i: (i, 0)),
            pl.BlockSpec((RT, 3), lambda i: (i, 0)),
            pl.BlockSpec((C, C), lambda i: (0, 0)),
            pl.BlockSpec((3, C), lambda i: (0, 0)),
            pl.BlockSpec((C, C), lambda i: (0, 0)),
            pl.BlockSpec((1, C), lambda i: (0, 0)),
        ],
        out_specs=[pl.BlockSpec((RT, C), lambda i: (i, 0)),
                   pl.BlockSpec((RT, C), lambda i: (i, 0))],
        out_shape=[jax.ShapeDtypeStruct((BN, C), f32),
                   jax.ShapeDtypeStruct((BN, C), f32)],
    )(x2, p2r, x1, p1r, wf2, wrel, wf1, b0r)

    T = 256
    p2t = jnp.transpose(points2, (0, 2, 1))
    idx = pl.pallas_call(
        functools.partial(_knn_kernel, n=N, k=K, t=T),
        grid=(B, N // T),
        in_specs=[
            pl.BlockSpec((1, T, 3), lambda b, i: (b, i, 0)),
            pl.BlockSpec((1, 3, N), lambda b, i: (b, 0, 0)),
        ],
        out_specs=pl.BlockSpec((1, T, K), lambda b, i: (b, i, 0)),
        out_shape=jax.ShapeDtypeStruct((B, N, K), jnp.int32),
    )(points1, p2t)

    idx3 = idx.reshape(_NW, R // (_NW * _CHUNK), _CHUNK)
    t2g3 = _sc_gather(idx3, t2).reshape(BN, K, C)

    GT = 256
    grid = (BN // GT,)
    in3 = pl.BlockSpec((GT, K, C), lambda i: (i, 0, 0))
    in2 = pl.BlockSpec((GT, C), lambda i: (i, 0))
    vec = pl.BlockSpec((1, C), lambda i: (0, 0))
    vec_shape = jax.ShapeDtypeStruct((1, C), f32)

    s0, ss0 = pl.pallas_call(
        _stats0_kernel,
        grid=grid,
        in_specs=[in3, in2],
        out_specs=[vec, vec],
        out_shape=[vec_shape, vec_shape],
    )(t2g3, f1b)

    def _affine(s, ss, gamma, beta):
        mean = s / R
        var = ss / R - mean * mean
        scale = gamma.reshape(1, C) / jnp.sqrt(var + _EPS)
        shift = beta.reshape(1, C) - mean * scale
        return scale, shift

    sc0, sh0 = _affine(s0, ss0, gamma0, beta0)

    y1, s1, ss1 = pl.pallas_call(
        functools.partial(_layer1_kernel, gt=GT, k=K, c=C),
        grid=grid,
        in_specs=[in3, in2, vec, vec,
                  pl.BlockSpec((C, C), lambda i: (0, 0)), vec],
        out_specs=[in3, vec, vec],
        out_shape=[jax.ShapeDtypeStruct((BN, K, C), jnp.bfloat16),
                   vec_shape, vec_shape],
    )(t2g3, f1b, sc0, sh0, jnp.transpose(W1).astype(jnp.bfloat16),
      b1.reshape(1, C))

    sc1, sh1 = _affine(s1, ss1, gamma1, beta1)

    y2, s2, ss2 = pl.pallas_call(
        functools.partial(_layer2_kernel, gt=GT, k=K, c=C),
        grid=grid,
        in_specs=[in3, vec, vec,
                  pl.BlockSpec((C, C), lambda i: (0, 0)), vec],
        out_specs=[in3, vec, vec],
        out_shape=[jax.ShapeDtypeStruct((BN, K, C), jnp.bfloat16),
                   vec_shape, vec_shape],
    )(y1, sc1, sh1, jnp.transpose(W2).astype(jnp.bfloat16),
      b2.reshape(1, C))

    sc2, sh2 = _affine(s2, ss2, gamma2, beta2)

    outr = pl.pallas_call(
        _final_kernel,
        grid=grid,
        in_specs=[in3, vec, vec],
        out_specs=in2,
        out_shape=jax.ShapeDtypeStruct((BN, C), f32),
    )(y2, sc2, sh2)

    return jnp.transpose(outr.reshape(B, N, C), (0, 2, 1))


# speedup vs baseline: 16.5182x; 1.0700x over previous
import functools

import jax
import jax.numpy as jnp
from jax import lax
from jax.experimental import pallas as pl
from jax.experimental.pallas import tpu as pltpu
from jax.experimental.pallas import tpu_sc as plsc

_EPS = 1e-3
_K = 16
_NC, _NS = 2, 16
_NW = _NC * _NS
_CHUNK = 128


def _tables_kernel(x2_ref, p2_ref, x1_ref, p1_ref, wf2_ref, wrel_ref,
                   wf1_ref, b0_ref, t2_ref, f1b_ref):
    t2_ref[...] = (
        jnp.dot(x2_ref[...], wf2_ref[...], preferred_element_type=jnp.float32)
        + jnp.dot(p2_ref[...], wrel_ref[...], preferred_element_type=jnp.float32))
    f1b_ref[...] = (
        jnp.dot(x1_ref[...], wf1_ref[...], preferred_element_type=jnp.float32)
        - jnp.dot(p1_ref[...], wrel_ref[...], preferred_element_type=jnp.float32)
        + b0_ref[...])


def _oddeven_merge(lo, hi, r):
    step = r * 2
    if step < hi - lo:
        yield from _oddeven_merge(lo, hi, step)
        yield from _oddeven_merge(lo + r, hi, step)
        yield from ((i, i + r) for i in range(lo + r, hi - r, step))
    else:
        yield (lo, lo + r)


def _oddeven_sort(lo, hi):
    if (hi - lo) >= 1:
        mid = lo + ((hi - lo) // 2)
        yield from _oddeven_sort(lo, mid)
        yield from _oddeven_sort(mid + 1, hi)
        yield from _oddeven_merge(lo, hi, 1)


_BATCHER16 = list(_oddeven_sort(0, 15))
_DEPTH = 6


def _knn_kernel(p1_ref, p2t_ref, idx_ref, *, n, k, t, off):
    q = p1_ref[0]
    pt = p2t_ref[0]
    qn = jnp.sum(q * q, axis=1, keepdims=True)
    pn = jnp.sum(pt * pt, axis=0, keepdims=True)
    dist = qn - 2.0 * jnp.dot(q, pt, preferred_element_type=jnp.float32) + pn
    g = 16
    l = n // g
    lane = lax.broadcasted_iota(jnp.int32, (t, l), 1)
    inf = jnp.float32(jnp.inf)
    big = jnp.int32(n)
    base = (pl.program_id(0) + off) * n

    svals = [dist[:, j * l:(j + 1) * l] for j in range(g)]
    sidx = [lane + (j * l) for j in range(g)]
    for a, b in _BATCHER16:
        swap = svals[a] > svals[b]
        va, vb = svals[a], svals[b]
        svals[a] = jnp.where(swap, vb, va)
        svals[b] = jnp.where(swap, va, vb)
        ia, ib = sidx[a], sidx[b]
        sidx[a] = jnp.where(swap, ib, ia)
        sidx[b] = jnp.where(swap, ia, ib)

    heads = svals[0]
    hflat = sidx[0]
    pcol = jnp.zeros((t, l), jnp.int32)
    outs = []
    for _ in range(k):
        m = jnp.min(heads, axis=1, keepdims=True)
        csel = jnp.where(heads == m, hflat, big)
        selflat = jnp.min(csel, axis=1, keepdims=True)
        outs.append(selflat)
        hit = csel == selflat
        pcol = pcol + hit.astype(jnp.int32)
        nh = jnp.full((t, l), inf)
        nf = jnp.full((t, l), big)
        for j in range(_DEPTH - 1, 0, -1):
            pj = pcol == j
            nh = jnp.where(pj, svals[j], nh)
            nf = jnp.where(pj, sidx[j], nf)
        heads = jnp.where(hit, nh, heads)
        hflat = jnp.where(hit, nf, hflat)

    idx_ref[0] = jnp.concatenate(outs, axis=1) + base

    @pl.when(jnp.any(pcol >= _DEPTH))
    def _():
        d = dist
        iota = lax.broadcasted_iota(jnp.int32, (t, n), 1)
        cols = []
        for _ in range(k):
            mm = jnp.min(d, axis=1, keepdims=True)
            cand = jnp.where(d == mm, iota, big)
            sel = jnp.min(cand, axis=1, keepdims=True)
            cols.append(sel)
            d = jnp.where(iota == sel, inf, d)
        idx_ref[0] = jnp.concatenate(cols, axis=1) + base


def _sc_gather(idx3, table):
    nw, nch, chunk = idx3.shape
    c = table.shape[1]
    bpw = nch * chunk
    mesh = plsc.VectorSubcoreMesh(core_axis_name="c", subcore_axis_name="s",
                                  num_cores=_NC, num_subcores=_NS)

    def body(idx_hbm, table_hbm, out_hbm, i0, i1, r0, r1, sg0, sg1, ss0, ss1):
        wid = lax.axis_index("s") * _NC + lax.axis_index("c")
        base = wid * bpw

        def do_pair(p, drain_prev):
            c0 = 2 * p
            pltpu.sync_copy(idx_hbm.at[wid, c0], i0)
            g0 = pltpu.async_copy(table_hbm.at[i0], r0, sg0)
            if drain_prev:
                pltpu.make_async_copy(
                    r1, out_hbm.at[pl.ds(base, chunk)], ss1).wait()
            g0.wait()
            s0 = pltpu.async_copy(
                r0, out_hbm.at[pl.ds(base + c0 * chunk, chunk)], ss0)
            pltpu.sync_copy(idx_hbm.at[wid, c0 + 1], i1)
            g1 = pltpu.async_copy(table_hbm.at[i1], r1, sg1)
            g1.wait()
            s0.wait()
            pltpu.async_copy(
                r1, out_hbm.at[pl.ds(base + (c0 + 1) * chunk, chunk)], ss1)

        do_pair(0, False)

        @pl.loop(1, nch // 2)
        def _pairs(p):
            do_pair(p, True)

        pltpu.make_async_copy(r1, out_hbm.at[pl.ds(base, chunk)], ss1).wait()

    fn = pl.kernel(
        body,
        out_type=jax.ShapeDtypeStruct((nw * bpw, c), jnp.float32),
        mesh=mesh,
        scratch_types=[
            pltpu.VMEM((chunk,), jnp.int32),
            pltpu.VMEM((chunk,), jnp.int32),
            pltpu.VMEM((chunk, c), jnp.float32),
            pltpu.VMEM((chunk, c), jnp.float32),
            pltpu.SemaphoreType.DMA,
            pltpu.SemaphoreType.DMA,
            pltpu.SemaphoreType.DMA,
            pltpu.SemaphoreType.DMA,
        ],
    )
    return fn(idx3, table)


def _stats0_kernel(t2g_ref, f1b_ref, s_ref, ss_ref):
    y = t2g_ref[...] + f1b_ref[...][:, None, :]
    s = jnp.sum(jnp.sum(y, axis=1), axis=0, keepdims=True)
    ss = jnp.sum(jnp.sum(y * y, axis=1), axis=0, keepdims=True)

    @pl.when(pl.program_id(0) == 0)
    def _():
        s_ref[...] = jnp.zeros_like(s_ref)
        ss_ref[...] = jnp.zeros_like(ss_ref)

    s_ref[...] += s
    ss_ref[...] += ss


def _layer1_kernel(t2g_ref, f1b_ref, sc_ref, sh_ref, w_ref, b_ref,
                   y_ref, s_ref, ss_ref, *, gt, k, c):
    y0 = t2g_ref[...] + f1b_ref[...][:, None, :]
    h = jnp.maximum(y0 * sc_ref[...][None] + sh_ref[...][None], 0.0)
    y = jnp.dot(h.reshape(gt * k, c).astype(jnp.bfloat16), w_ref[...],
                preferred_element_type=jnp.float32) + b_ref[...]
    y_ref[...] = y.reshape(gt, k, c).astype(jnp.bfloat16)
    s = jnp.sum(y, axis=0, keepdims=True)
    ss = jnp.sum(y * y, axis=0, keepdims=True)

    @pl.when(pl.program_id(0) == 0)
    def _():
        s_ref[...] = jnp.zeros_like(s_ref)
        ss_ref[...] = jnp.zeros_like(ss_ref)

    s_ref[...] += s
    ss_ref[...] += ss


def _layer2_kernel(yin_ref, sc_ref, sh_ref, w_ref, b_ref,
                   y_ref, s_ref, ss_ref, *, gt, k, c):
    yin = yin_ref[...].astype(jnp.float32)
    h = jnp.maximum(yin * sc_ref[...][None] + sh_ref[...][None], 0.0)
    y = jnp.dot(h.reshape(gt * k, c).astype(jnp.bfloat16), w_ref[...],
                preferred_element_type=jnp.float32) + b_ref[...]
    y_ref[...] = y.reshape(gt, k, c).astype(jnp.bfloat16)
    s = jnp.sum(y, axis=0, keepdims=True)
    ss = jnp.sum(y * y, axis=0, keepdims=True)

    @pl.when(pl.program_id(0) == 0)
    def _():
        s_ref[...] = jnp.zeros_like(s_ref)
        ss_ref[...] = jnp.zeros_like(ss_ref)

    s_ref[...] += s
    ss_ref[...] += ss


def _final_kernel(yin_ref, sc_ref, sh_ref, out_ref):
    yin = yin_ref[...].astype(jnp.float32)
    h = jnp.maximum(yin * sc_ref[...][None] + sh_ref[...][None], 0.0)
    out_ref[...] = jnp.max(h, axis=1)


def kernel(points1, points2, features1, features2,
           W0, b0, gamma0, beta0,
           W1, b1, gamma1, beta1,
           W2, b2, gamma2, beta2):
    B, N, _ = points1.shape
    C = features1.shape[1]
    K = _K
    BN = B * N
    R = BN * K
    f32 = jnp.float32

    x2 = jnp.transpose(features2, (0, 2, 1)).reshape(BN, C)
    x1 = jnp.transpose(features1, (0, 2, 1)).reshape(BN, C)
    p2r = points2.reshape(BN, 3)
    p1r = points1.reshape(BN, 3)
    wrel = jnp.transpose(W0[:, :3])
    wf2 = jnp.transpose(W0[:, 3:3 + C])
    wf1 = jnp.transpose(W0[:, 3 + C:])
    b0r = b0.reshape(1, C)

    RT = 2048
    t2, f1b = pl.pallas_call(
        _tables_kernel,
        grid=(BN // RT,),
        in_specs=[
            pl.BlockSpec((RT, C), lambda i: (i, 0)),
            pl.BlockSpec((RT, 3), lambda i: (i, 0)),
            pl.BlockSpec((RT, C), lambda i: (i, 0)),
            pl.BlockSpec((RT, 3), lambda i: (i, 0)),
            pl.BlockSpec((C, C), lambda i: (0, 0)),
            pl.BlockSpec((3, C), lambda i: (0, 0)),
            pl.BlockSpec((C, C), lambda i: (0, 0)),
            pl.BlockSpec((1, C), lambda i: (0, 0)),
        ],
        out_specs=[pl.BlockSpec((RT, C), lambda i: (i, 0)),
                   pl.BlockSpec((RT, C), lambda i: (i, 0))],
        out_shape=[jax.ShapeDtypeStruct((BN, C), f32),
                   jax.ShapeDtypeStruct((BN, C), f32)],
    )(x2, p2r, x1, p1r, wf2, wrel, wf1, b0r)

    T = 256
    p2t = jnp.transpose(points2, (0, 2, 1))
    HB = B // 2
    HBN = HB * N
    GT = 256
    grid = (HBN // GT,)
    in3 = pl.BlockSpec((GT, K, C), lambda i: (i, 0, 0))
    in2 = pl.BlockSpec((GT, C), lambda i: (i, 0))
    vec = pl.BlockSpec((1, C), lambda i: (0, 0))
    vec_shape = jax.ShapeDtypeStruct((1, C), f32)

    def knn_half(h):
        return pl.pallas_call(
            functools.partial(_knn_kernel, n=N, k=K, t=T, off=h * HB),
            grid=(HB, N // T),
            in_specs=[
                pl.BlockSpec((1, T, 3), lambda b, i: (b, i, 0)),
                pl.BlockSpec((1, 3, N), lambda b, i: (b, 0, 0)),
            ],
            out_specs=pl.BlockSpec((1, T, K), lambda b, i: (b, i, 0)),
            out_shape=jax.ShapeDtypeStruct((HB, N, K), jnp.int32),
        )(points1[h * HB:(h + 1) * HB], p2t[h * HB:(h + 1) * HB])

    idx_h = [knn_half(0), knn_half(1)]
    t2g_h = [_sc_gather(ix.reshape(_NW, HBN * K // (_NW * _CHUNK), _CHUNK),
                        t2).reshape(HBN, K, C) for ix in idx_h]
    f1b_h = [f1b[:HBN], f1b[HBN:]]

    def stats0_half(h):
        return pl.pallas_call(
            _stats0_kernel,
            grid=grid,
            in_specs=[in3, in2],
            out_specs=[vec, vec],
            out_shape=[vec_shape, vec_shape],
        )(t2g_h[h], f1b_h[h])

    st0 = [stats0_half(0), stats0_half(1)]
    s0, ss0 = st0[0][0] + st0[1][0], st0[0][1] + st0[1][1]

    def _affine(s, ss, gamma, beta):
        mean = s / R
        var = ss / R - mean * mean
        scale = gamma.reshape(1, C) / jnp.sqrt(var + _EPS)
        shift = beta.reshape(1, C) - mean * scale
        return scale, shift

    sc0, sh0 = _affine(s0, ss0, gamma0, beta0)
    w1t = jnp.transpose(W1).astype(jnp.bfloat16)
    b1r = b1.reshape(1, C)

    def layer1_half(h):
        return pl.pallas_call(
            functools.partial(_layer1_kernel, gt=GT, k=K, c=C),
            grid=grid,
            in_specs=[in3, in2, vec, vec,
                      pl.BlockSpec((C, C), lambda i: (0, 0)), vec],
            out_specs=[in3, vec, vec],
            out_shape=[jax.ShapeDtypeStruct((HBN, K, C), jnp.bfloat16),
                       vec_shape, vec_shape],
        )(t2g_h[h], f1b_h[h], sc0, sh0, w1t, b1r)

    l1 = [layer1_half(0), layer1_half(1)]
    sc1, sh1 = _affine(l1[0][1] + l1[1][1], l1[0][2] + l1[1][2],
                       gamma1, beta1)
    w2t = jnp.transpose(W2).astype(jnp.bfloat16)
    b2r = b2.reshape(1, C)

    def layer2_half(h):
        return pl.pallas_call(
            functools.partial(_layer2_kernel, gt=GT, k=K, c=C),
            grid=grid,
            in_specs=[in3, vec, vec,
                      pl.BlockSpec((C, C), lambda i: (0, 0)), vec],
            out_specs=[in3, vec, vec],
            out_shape=[jax.ShapeDtypeStruct((HBN, K, C), jnp.bfloat16),
                       vec_shape, vec_shape],
        )(l1[h][0], sc1, sh1, w2t, b2r)

    l2 = [layer2_half(0), layer2_half(1)]
    sc2, sh2 = _affine(l2[0][1] + l2[1][1], l2[0][2] + l2[1][2],
                       gamma2, beta2)

    def final_half(h):
        return pl.pallas_call(
            _final_kernel,
            grid=grid,
            in_specs=[in3, vec, vec],
            out_specs=in2,
            out_shape=jax.ShapeDtypeStruct((HBN, C), f32),
        )(l2[h][0], sc2, sh2)

    outr = jnp.concatenate([final_half(0), final_half(1)], axis=0)
    return jnp.transpose(outr.reshape(B, N, C), (0, 2, 1))


# speedup vs baseline: 16.6150x; 1.0059x over previous
import functools

import jax
import jax.numpy as jnp
from jax import lax
from jax.experimental import pallas as pl
from jax.experimental.pallas import tpu as pltpu
from jax.experimental.pallas import tpu_sc as plsc

_EPS = 1e-3
_K = 16
_NC, _NS = 2, 16
_NW = _NC * _NS
_CHUNK = 128


def _tables_kernel(x2_ref, p2_ref, x1_ref, p1_ref, wf2_ref, wrel_ref,
                   wf1_ref, b0_ref, t2_ref, f1b_ref):
    t2_ref[...] = (
        jnp.dot(x2_ref[...], wf2_ref[...], preferred_element_type=jnp.float32)
        + jnp.dot(p2_ref[...], wrel_ref[...], preferred_element_type=jnp.float32))
    f1b_ref[...] = (
        jnp.dot(x1_ref[...], wf1_ref[...], preferred_element_type=jnp.float32)
        - jnp.dot(p1_ref[...], wrel_ref[...], preferred_element_type=jnp.float32)
        + b0_ref[...])


def _oddeven_merge(lo, hi, r):
    step = r * 2
    if step < hi - lo:
        yield from _oddeven_merge(lo, hi, step)
        yield from _oddeven_merge(lo + r, hi, step)
        yield from ((i, i + r) for i in range(lo + r, hi - r, step))
    else:
        yield (lo, lo + r)


def _oddeven_sort(lo, hi):
    if (hi - lo) >= 1:
        mid = lo + ((hi - lo) // 2)
        yield from _oddeven_sort(lo, mid)
        yield from _oddeven_sort(mid + 1, hi)
        yield from _oddeven_merge(lo, hi, 1)


_BATCHER16 = list(_oddeven_sort(0, 15))
_DEPTH = 6


def _knn_kernel(p1_ref, p2t_ref, idx_ref, *, n, k, t, off):
    q = p1_ref[0]
    pt = p2t_ref[0]
    qn = jnp.sum(q * q, axis=1, keepdims=True)
    pn = jnp.sum(pt * pt, axis=0, keepdims=True)
    dist = qn - 2.0 * jnp.dot(q, pt, preferred_element_type=jnp.float32) + pn
    g = 16
    l = n // g
    lane = lax.broadcasted_iota(jnp.int32, (t, l), 1)
    inf = jnp.float32(jnp.inf)
    big = jnp.int32(n)
    base = (pl.program_id(0) + off) * n

    svals = [dist[:, j * l:(j + 1) * l] for j in range(g)]
    sidx = [lane + (j * l) for j in range(g)]
    for a, b in _BATCHER16:
        swap = svals[a] > svals[b]
        va, vb = svals[a], svals[b]
        svals[a] = jnp.where(swap, vb, va)
        svals[b] = jnp.where(swap, va, vb)
        ia, ib = sidx[a], sidx[b]
        sidx[a] = jnp.where(swap, ib, ia)
        sidx[b] = jnp.where(swap, ia, ib)

    heads = svals[0]
    hflat = sidx[0]
    pcol = jnp.zeros((t, l), jnp.int32)
    outs = []
    for _ in range(k):
        m = jnp.min(heads, axis=1, keepdims=True)
        csel = jnp.where(heads == m, hflat, big)
        selflat = jnp.min(csel, axis=1, keepdims=True)
        outs.append(selflat)
        hit = csel == selflat
        pcol = pcol + hit.astype(jnp.int32)
        nh = jnp.full((t, l), inf)
        nf = jnp.full((t, l), big)
        for j in range(_DEPTH - 1, 0, -1):
            pj = pcol == j
            nh = jnp.where(pj, svals[j], nh)
            nf = jnp.where(pj, sidx[j], nf)
        heads = jnp.where(hit, nh, heads)
        hflat = jnp.where(hit, nf, hflat)

    idx_ref[0] = jnp.concatenate(outs, axis=1) + base

    @pl.when(jnp.any(pcol >= _DEPTH))
    def _():
        d = dist
        iota = lax.broadcasted_iota(jnp.int32, (t, n), 1)
        cols = []
        for _ in range(k):
            mm = jnp.min(d, axis=1, keepdims=True)
            cand = jnp.where(d == mm, iota, big)
            sel = jnp.min(cand, axis=1, keepdims=True)
            cols.append(sel)
            d = jnp.where(iota == sel, inf, d)
        idx_ref[0] = jnp.concatenate(cols, axis=1) + base


def _sc_gather(idx3, table):
    nw, nch, chunk = idx3.shape
    c = table.shape[1]
    bpw = nch * chunk
    mesh = plsc.VectorSubcoreMesh(core_axis_name="c", subcore_axis_name="s",
                                  num_cores=_NC, num_subcores=_NS)

    def body(idx_hbm, table_hbm, out_hbm, i0, i1, r0, r1, sg0, sg1, ss0, ss1):
        wid = lax.axis_index("s") * _NC + lax.axis_index("c")
        base = wid * bpw

        def do_pair(p, drain_prev):
            c0 = 2 * p
            pltpu.sync_copy(idx_hbm.at[wid, c0], i0)
            g0 = pltpu.async_copy(table_hbm.at[i0], r0, sg0)
            if drain_prev:
                pltpu.make_async_copy(
                    r1, out_hbm.at[pl.ds(base, chunk)], ss1).wait()
            g0.wait()
            s0 = pltpu.async_copy(
                r0, out_hbm.at[pl.ds(base + c0 * chunk, chunk)], ss0)
            pltpu.sync_copy(idx_hbm.at[wid, c0 + 1], i1)
            g1 = pltpu.async_copy(table_hbm.at[i1], r1, sg1)
            g1.wait()
            s0.wait()
            pltpu.async_copy(
                r1, out_hbm.at[pl.ds(base + (c0 + 1) * chunk, chunk)], ss1)

        do_pair(0, False)

        @pl.loop(1, nch // 2)
        def _pairs(p):
            do_pair(p, True)

        pltpu.make_async_copy(r1, out_hbm.at[pl.ds(base, chunk)], ss1).wait()

    fn = pl.kernel(
        body,
        out_type=jax.ShapeDtypeStruct((nw * bpw, c), jnp.float32),
        mesh=mesh,
        scratch_types=[
            pltpu.VMEM((chunk,), jnp.int32),
            pltpu.VMEM((chunk,), jnp.int32),
            pltpu.VMEM((chunk, c), jnp.float32),
            pltpu.VMEM((chunk, c), jnp.float32),
            pltpu.SemaphoreType.DMA,
            pltpu.SemaphoreType.DMA,
            pltpu.SemaphoreType.DMA,
            pltpu.SemaphoreType.DMA,
        ],
    )
    return fn(idx3, table)


def _stats0_kernel(t2g_ref, f1b_ref, s_ref, ss_ref):
    y = t2g_ref[...] + f1b_ref[...][:, None, :]
    s = jnp.sum(jnp.sum(y, axis=1), axis=0, keepdims=True)
    ss = jnp.sum(jnp.sum(y * y, axis=1), axis=0, keepdims=True)

    @pl.when(pl.program_id(0) == 0)
    def _():
        s_ref[...] = jnp.zeros_like(s_ref)
        ss_ref[...] = jnp.zeros_like(ss_ref)

    s_ref[...] += s
    ss_ref[...] += ss


def _layer1_kernel(t2g_ref, f1b_ref, sc_ref, sh_ref, w_ref, b_ref,
                   y_ref, s_ref, ss_ref, *, gt, k, c):
    y0 = t2g_ref[...] + f1b_ref[...][:, None, :]
    h = jnp.maximum(y0 * sc_ref[...][None] + sh_ref[...][None], 0.0)
    y = jnp.dot(h.reshape(gt * k, c).astype(jnp.bfloat16), w_ref[...],
                preferred_element_type=jnp.float32) + b_ref[...]
    y_ref[...] = y.reshape(gt, k, c).astype(jnp.bfloat16)
    s = jnp.sum(y, axis=0, keepdims=True)
    ss = jnp.sum(y * y, axis=0, keepdims=True)

    @pl.when(pl.program_id(0) == 0)
    def _():
        s_ref[...] = jnp.zeros_like(s_ref)
        ss_ref[...] = jnp.zeros_like(ss_ref)

    s_ref[...] += s
    ss_ref[...] += ss


def _layer2_kernel(yin_ref, sc_ref, sh_ref, w_ref, b_ref,
                   y_ref, s_ref, ss_ref, *, gt, k, c):
    yin = yin_ref[...].astype(jnp.float32)
    h = jnp.maximum(yin * sc_ref[...][None] + sh_ref[...][None], 0.0)
    y = jnp.dot(h.reshape(gt * k, c).astype(jnp.bfloat16), w_ref[...],
                preferred_element_type=jnp.float32) + b_ref[...]
    y_ref[...] = y.reshape(gt, k, c).astype(jnp.bfloat16)
    s = jnp.sum(y, axis=0, keepdims=True)
    ss = jnp.sum(y * y, axis=0, keepdims=True)

    @pl.when(pl.program_id(0) == 0)
    def _():
        s_ref[...] = jnp.zeros_like(s_ref)
        ss_ref[...] = jnp.zeros_like(ss_ref)

    s_ref[...] += s
    ss_ref[...] += ss


def _final_kernel(yin_ref, sc_ref, sh_ref, out_ref):
    yin = yin_ref[...].astype(jnp.float32)
    h = jnp.maximum(yin * sc_ref[...][None] + sh_ref[...][None], 0.0)
    out_ref[...] = jnp.max(h, axis=1)


def kernel(points1, points2, features1, features2,
           W0, b0, gamma0, beta0,
           W1, b1, gamma1, beta1,
           W2, b2, gamma2, beta2):
    B, N, _ = points1.shape
    C = features1.shape[1]
    K = _K
    BN = B * N
    R = BN * K
    f32 = jnp.float32

    x2 = jnp.transpose(features2, (0, 2, 1)).reshape(BN, C)
    x1 = jnp.transpose(features1, (0, 2, 1)).reshape(BN, C)
    p2r = points2.reshape(BN, 3)
    p1r = points1.reshape(BN, 3)
    wrel = jnp.transpose(W0[:, :3])
    wf2 = jnp.transpose(W0[:, 3:3 + C])
    wf1 = jnp.transpose(W0[:, 3 + C:])
    b0r = b0.reshape(1, C)

    RT = 2048
    t2, f1b = pl.pallas_call(
        _tables_kernel,
        grid=(BN // RT,),
        in_specs=[
            pl.BlockSpec((RT, C), lambda i: (i, 0)),
            pl.BlockSpec((RT, 3), lambda i: (i, 0)),
            pl.BlockSpec((RT, C), lambda i: (i, 0)),
            pl.BlockSpec((RT, 3), lambda i: (i, 0)),
            pl.BlockSpec((C, C), lambda i: (0, 0)),
            pl.BlockSpec((3, C), lambda i: (0, 0)),
            pl.BlockSpec((C, C), lambda i: (0, 0)),
            pl.BlockSpec((1, C), lambda i: (0, 0)),
        ],
        out_specs=[pl.BlockSpec((RT, C), lambda i: (i, 0)),
                   pl.BlockSpec((RT, C), lambda i: (i, 0))],
        out_shape=[jax.ShapeDtypeStruct((BN, C), f32),
                   jax.ShapeDtypeStruct((BN, C), f32)],
    )(x2, p2r, x1, p1r, wf2, wrel, wf1, b0r)

    T = 256
    p2t = jnp.transpose(points2, (0, 2, 1))
    HB = B // 4
    HBN = HB * N
    GT = 256
    grid = (HBN // GT,)
    in3 = pl.BlockSpec((GT, K, C), lambda i: (i, 0, 0))
    in2 = pl.BlockSpec((GT, C), lambda i: (i, 0))
    vec = pl.BlockSpec((1, C), lambda i: (0, 0))
    vec_shape = jax.ShapeDtypeStruct((1, C), f32)

    def knn_half(h):
        return pl.pallas_call(
            functools.partial(_knn_kernel, n=N, k=K, t=T, off=h * HB),
            grid=(HB, N // T),
            in_specs=[
                pl.BlockSpec((1, T, 3), lambda b, i: (b, i, 0)),
                pl.BlockSpec((1, 3, N), lambda b, i: (b, 0, 0)),
            ],
            out_specs=pl.BlockSpec((1, T, K), lambda b, i: (b, i, 0)),
            out_shape=jax.ShapeDtypeStruct((HB, N, K), jnp.int32),
        )(points1[h * HB:(h + 1) * HB], p2t[h * HB:(h + 1) * HB])

    NH = B // HB
    idx_h = [knn_half(h) for h in range(NH)]
    t2g_h = [_sc_gather(ix.reshape(_NW, HBN * K // (_NW * _CHUNK), _CHUNK),
                        t2).reshape(HBN, K, C) for ix in idx_h]
    f1b_h = [f1b[h * HBN:(h + 1) * HBN] for h in range(NH)]

    def stats0_half(h):
        return pl.pallas_call(
            _stats0_kernel,
            grid=grid,
            in_specs=[in3, in2],
            out_specs=[vec, vec],
            out_shape=[vec_shape, vec_shape],
        )(t2g_h[h], f1b_h[h])

    def _accum(parts, i):
        tot = parts[0][i]
        for p in parts[1:]:
            tot = tot + p[i]
        return tot

    st0 = [stats0_half(h) for h in range(NH)]
    s0, ss0 = _accum(st0, 0), _accum(st0, 1)

    def _affine(s, ss, gamma, beta):
        mean = s / R
        var = ss / R - mean * mean
        scale = gamma.reshape(1, C) / jnp.sqrt(var + _EPS)
        shift = beta.reshape(1, C) - mean * scale
        return scale, shift

    sc0, sh0 = _affine(s0, ss0, gamma0, beta0)
    w1t = jnp.transpose(W1).astype(jnp.bfloat16)
    b1r = b1.reshape(1, C)

    def layer1_half(h):
        return pl.pallas_call(
            functools.partial(_layer1_kernel, gt=GT, k=K, c=C),
            grid=grid,
            in_specs=[in3, in2, vec, vec,
                      pl.BlockSpec((C, C), lambda i: (0, 0)), vec],
            out_specs=[in3, vec, vec],
            out_shape=[jax.ShapeDtypeStruct((HBN, K, C), jnp.bfloat16),
                       vec_shape, vec_shape],
        )(t2g_h[h], f1b_h[h], sc0, sh0, w1t, b1r)

    l1 = [layer1_half(h) for h in range(NH)]
    sc1, sh1 = _affine(_accum(l1, 1), _accum(l1, 2), gamma1, beta1)
    w2t = jnp.transpose(W2).astype(jnp.bfloat16)
    b2r = b2.reshape(1, C)

    def layer2_half(h):
        return pl.pallas_call(
            functools.partial(_layer2_kernel, gt=GT, k=K, c=C),
            grid=grid,
            in_specs=[in3, vec, vec,
                      pl.BlockSpec((C, C), lambda i: (0, 0)), vec],
            out_specs=[in3, vec, vec],
            out_shape=[jax.ShapeDtypeStruct((HBN, K, C), jnp.bfloat16),
                       vec_shape, vec_shape],
        )(l1[h][0], sc1, sh1, w2t, b2r)

    l2 = [layer2_half(h) for h in range(NH)]
    sc2, sh2 = _affine(_accum(l2, 1), _accum(l2, 2), gamma2, beta2)

    def final_half(h):
        return pl.pallas_call(
            _final_kernel,
            grid=grid,
            in_specs=[in3, vec, vec],
            out_specs=in2,
            out_shape=jax.ShapeDtypeStruct((HBN, C), f32),
        )(l2[h][0], sc2, sh2)

    outr = jnp.concatenate([final_half(h) for h in range(NH)], axis=0)
    return jnp.transpose(outr.reshape(B, N, C), (0, 2, 1))


# speedup vs baseline: 17.0141x; 1.0240x over previous
import functools

import jax
import jax.numpy as jnp
from jax import lax
from jax.experimental import pallas as pl
from jax.experimental.pallas import tpu as pltpu
from jax.experimental.pallas import tpu_sc as plsc

_EPS = 1e-3
_K = 16
_NC, _NS = 2, 16
_NW = _NC * _NS
_CHUNK = 128


def _tables_kernel(x2_ref, p2_ref, x1_ref, p1_ref, wf2_ref, wrel_ref,
                   wf1_ref, b0_ref, t2_ref, f1b_ref):
    t2_ref[...] = (
        jnp.dot(x2_ref[...], wf2_ref[...], preferred_element_type=jnp.float32)
        + jnp.dot(p2_ref[...], wrel_ref[...], preferred_element_type=jnp.float32))
    f1b_ref[...] = (
        jnp.dot(x1_ref[...], wf1_ref[...], preferred_element_type=jnp.float32)
        - jnp.dot(p1_ref[...], wrel_ref[...], preferred_element_type=jnp.float32)
        + b0_ref[...])


def _oddeven_merge(lo, hi, r):
    step = r * 2
    if step < hi - lo:
        yield from _oddeven_merge(lo, hi, step)
        yield from _oddeven_merge(lo + r, hi, step)
        yield from ((i, i + r) for i in range(lo + r, hi - r, step))
    else:
        yield (lo, lo + r)


def _oddeven_sort(lo, hi):
    if (hi - lo) >= 1:
        mid = lo + ((hi - lo) // 2)
        yield from _oddeven_sort(lo, mid)
        yield from _oddeven_sort(mid + 1, hi)
        yield from _oddeven_merge(lo, hi, 1)


_DEPTH = 6


def _prune_to_selection(net, depth):
    needed = set(range(depth))
    kept = []
    for a, b in reversed(net):
        if a in needed or b in needed:
            kept.append((a, b))
            needed.add(a)
            needed.add(b)
    kept.reverse()
    return kept


_BATCHER16 = _prune_to_selection(list(_oddeven_sort(0, 15)), _DEPTH)


def _knn_kernel(p1_ref, p2t_ref, idx_ref, *, n, k, t, off):
    q = p1_ref[0]
    pt = p2t_ref[0]
    qn = jnp.sum(q * q, axis=1, keepdims=True)
    pn = jnp.sum(pt * pt, axis=0, keepdims=True)
    dist = qn - 2.0 * jnp.dot(q, pt, preferred_element_type=jnp.float32) + pn
    g = 16
    l = n // g
    lane = lax.broadcasted_iota(jnp.int32, (t, l), 1)
    inf = jnp.float32(jnp.inf)
    big = jnp.int32(n)
    base = (pl.program_id(0) + off) * n

    svals = [dist[:, j * l:(j + 1) * l] for j in range(g)]
    sidx = [lane + (j * l) for j in range(g)]
    for a, b in _BATCHER16:
        swap = svals[a] > svals[b]
        va, vb = svals[a], svals[b]
        svals[a] = jnp.where(swap, vb, va)
        svals[b] = jnp.where(swap, va, vb)
        ia, ib = sidx[a], sidx[b]
        sidx[a] = jnp.where(swap, ib, ia)
        sidx[b] = jnp.where(swap, ia, ib)

    heads = svals[0]
    hflat = sidx[0]
    pcol = jnp.zeros((t, l), jnp.int32)
    outs = []
    for _ in range(k):
        m = jnp.min(heads, axis=1, keepdims=True)
        csel = jnp.where(heads == m, hflat, big)
        selflat = jnp.min(csel, axis=1, keepdims=True)
        outs.append(selflat)
        hit = csel == selflat
        pcol = pcol + hit.astype(jnp.int32)
        nh = jnp.full((t, l), inf)
        nf = jnp.full((t, l), big)
        for j in range(_DEPTH - 1, 0, -1):
            pj = pcol == j
            nh = jnp.where(pj, svals[j], nh)
            nf = jnp.where(pj, sidx[j], nf)
        heads = jnp.where(hit, nh, heads)
        hflat = jnp.where(hit, nf, hflat)

    idx_ref[0] = jnp.concatenate(outs, axis=1) + base

    @pl.when(jnp.any(pcol >= _DEPTH))
    def _():
        d = dist
        iota = lax.broadcasted_iota(jnp.int32, (t, n), 1)
        cols = []
        for _ in range(k):
            mm = jnp.min(d, axis=1, keepdims=True)
            cand = jnp.where(d == mm, iota, big)
            sel = jnp.min(cand, axis=1, keepdims=True)
            cols.append(sel)
            d = jnp.where(iota == sel, inf, d)
        idx_ref[0] = jnp.concatenate(cols, axis=1) + base


def _sc_gather(idx3, table):
    nw, nch, chunk = idx3.shape
    c = table.shape[1]
    bpw = nch * chunk
    mesh = plsc.VectorSubcoreMesh(core_axis_name="c", subcore_axis_name="s",
                                  num_cores=_NC, num_subcores=_NS)

    def body(idx_hbm, table_hbm, out_hbm, i0, i1, r0, r1, sg0, sg1, ss0, ss1):
        wid = lax.axis_index("s") * _NC + lax.axis_index("c")
        base = wid * bpw

        def do_pair(p, drain_prev):
            c0 = 2 * p
            pltpu.sync_copy(idx_hbm.at[wid, c0], i0)
            g0 = pltpu.async_copy(table_hbm.at[i0], r0, sg0)
            if drain_prev:
                pltpu.make_async_copy(
                    r1, out_hbm.at[pl.ds(base, chunk)], ss1).wait()
            g0.wait()
            s0 = pltpu.async_copy(
                r0, out_hbm.at[pl.ds(base + c0 * chunk, chunk)], ss0)
            pltpu.sync_copy(idx_hbm.at[wid, c0 + 1], i1)
            g1 = pltpu.async_copy(table_hbm.at[i1], r1, sg1)
            g1.wait()
            s0.wait()
            pltpu.async_copy(
                r1, out_hbm.at[pl.ds(base + (c0 + 1) * chunk, chunk)], ss1)

        do_pair(0, False)

        @pl.loop(1, nch // 2)
        def _pairs(p):
            do_pair(p, True)

        pltpu.make_async_copy(r1, out_hbm.at[pl.ds(base, chunk)], ss1).wait()

    fn = pl.kernel(
        body,
        out_type=jax.ShapeDtypeStruct((nw * bpw, c), jnp.float32),
        mesh=mesh,
        scratch_types=[
            pltpu.VMEM((chunk,), jnp.int32),
            pltpu.VMEM((chunk,), jnp.int32),
            pltpu.VMEM((chunk, c), jnp.float32),
            pltpu.VMEM((chunk, c), jnp.float32),
            pltpu.SemaphoreType.DMA,
            pltpu.SemaphoreType.DMA,
            pltpu.SemaphoreType.DMA,
            pltpu.SemaphoreType.DMA,
        ],
    )
    return fn(idx3, table)


def _stats0_kernel(t2g_ref, f1b_ref, s_ref, ss_ref):
    y = t2g_ref[...] + f1b_ref[...][:, None, :]
    s = jnp.sum(jnp.sum(y, axis=1), axis=0, keepdims=True)
    ss = jnp.sum(jnp.sum(y * y, axis=1), axis=0, keepdims=True)

    @pl.when(pl.program_id(0) == 0)
    def _():
        s_ref[...] = jnp.zeros_like(s_ref)
        ss_ref[...] = jnp.zeros_like(ss_ref)

    s_ref[...] += s
    ss_ref[...] += ss


def _layer1_kernel(t2g_ref, f1b_ref, sc_ref, sh_ref, w_ref, b_ref,
                   y_ref, s_ref, ss_ref, *, gt, k, c):
    y0 = t2g_ref[...] + f1b_ref[...][:, None, :]
    h = jnp.maximum(y0 * sc_ref[...][None] + sh_ref[...][None], 0.0)
    y = jnp.dot(h.reshape(gt * k, c).astype(jnp.bfloat16), w_ref[...],
                preferred_element_type=jnp.float32) + b_ref[...]
    y_ref[...] = y.reshape(gt, k, c).astype(jnp.bfloat16)
    s = jnp.sum(y, axis=0, keepdims=True)
    ss = jnp.sum(y * y, axis=0, keepdims=True)

    @pl.when(pl.program_id(0) == 0)
    def _():
        s_ref[...] = jnp.zeros_like(s_ref)
        ss_ref[...] = jnp.zeros_like(ss_ref)

    s_ref[...] += s
    ss_ref[...] += ss


def _layer2_kernel(yin_ref, sc_ref, sh_ref, w_ref, b_ref,
                   ymax_ref, ymin_ref, s_ref, ss_ref, *, gt, k, c):
    yin = yin_ref[...].astype(jnp.float32)
    h = jnp.maximum(yin * sc_ref[...][None] + sh_ref[...][None], 0.0)
    y = jnp.dot(h.reshape(gt * k, c).astype(jnp.bfloat16), w_ref[...],
                preferred_element_type=jnp.float32) + b_ref[...]
    y3 = y.reshape(gt, k, c)
    ymax_ref[...] = jnp.max(y3, axis=1)
    ymin_ref[...] = jnp.min(y3, axis=1)
    s = jnp.sum(y, axis=0, keepdims=True)
    ss = jnp.sum(y * y, axis=0, keepdims=True)

    @pl.when(pl.program_id(0) == 0)
    def _():
        s_ref[...] = jnp.zeros_like(s_ref)
        ss_ref[...] = jnp.zeros_like(ss_ref)

    s_ref[...] += s
    ss_ref[...] += ss


def _final_kernel(ymax_ref, ymin_ref, sc_ref, sh_ref, out_ref):
    sc = sc_ref[...]
    sh = sh_ref[...]
    ext = jnp.where(sc >= 0.0, ymax_ref[...], ymin_ref[...])
    out_ref[...] = jnp.maximum(ext * sc + sh, 0.0)


def kernel(points1, points2, features1, features2,
           W0, b0, gamma0, beta0,
           W1, b1, gamma1, beta1,
           W2, b2, gamma2, beta2):
    B, N, _ = points1.shape
    C = features1.shape[1]
    K = _K
    BN = B * N
    R = BN * K
    f32 = jnp.float32

    x2 = jnp.transpose(features2, (0, 2, 1)).reshape(BN, C)
    x1 = jnp.transpose(features1, (0, 2, 1)).reshape(BN, C)
    p2r = points2.reshape(BN, 3)
    p1r = points1.reshape(BN, 3)
    wrel = jnp.transpose(W0[:, :3])
    wf2 = jnp.transpose(W0[:, 3:3 + C])
    wf1 = jnp.transpose(W0[:, 3 + C:])
    b0r = b0.reshape(1, C)

    RT = 2048
    t2, f1b = pl.pallas_call(
        _tables_kernel,
        grid=(BN // RT,),
        in_specs=[
            pl.BlockSpec((RT, C), lambda i: (i, 0)),
            pl.BlockSpec((RT, 3), lambda i: (i, 0)),
            pl.BlockSpec((RT, C), lambda i: (i, 0)),
            pl.BlockSpec((RT, 3), lambda i: (i, 0)),
            pl.BlockSpec((C, C), lambda i: (0, 0)),
            pl.BlockSpec((3, C), lambda i: (0, 0)),
            pl.BlockSpec((C, C), lambda i: (0, 0)),
            pl.BlockSpec((1, C), lambda i: (0, 0)),
        ],
        out_specs=[pl.BlockSpec((RT, C), lambda i: (i, 0)),
                   pl.BlockSpec((RT, C), lambda i: (i, 0))],
        out_shape=[jax.ShapeDtypeStruct((BN, C), f32),
                   jax.ShapeDtypeStruct((BN, C), f32)],
    )(x2, p2r, x1, p1r, wf2, wrel, wf1, b0r)

    T = 256
    p2t = jnp.transpose(points2, (0, 2, 1))
    HB = B // 4
    HBN = HB * N
    GT = 256
    grid = (HBN // GT,)
    in3 = pl.BlockSpec((GT, K, C), lambda i: (i, 0, 0))
    in2 = pl.BlockSpec((GT, C), lambda i: (i, 0))
    vec = pl.BlockSpec((1, C), lambda i: (0, 0))
    vec_shape = jax.ShapeDtypeStruct((1, C), f32)

    def knn_half(h):
        return pl.pallas_call(
            functools.partial(_knn_kernel, n=N, k=K, t=T, off=h * HB),
            grid=(HB, N // T),
            in_specs=[
                pl.BlockSpec((1, T, 3), lambda b, i: (b, i, 0)),
                pl.BlockSpec((1, 3, N), lambda b, i: (b, 0, 0)),
            ],
            out_specs=pl.BlockSpec((1, T, K), lambda b, i: (b, i, 0)),
            out_shape=jax.ShapeDtypeStruct((HB, N, K), jnp.int32),
        )(points1[h * HB:(h + 1) * HB], p2t[h * HB:(h + 1) * HB])

    NH = B // HB
    idx_h = [knn_half(h) for h in range(NH)]
    t2g_h = [_sc_gather(ix.reshape(_NW, HBN * K // (_NW * _CHUNK), _CHUNK),
                        t2).reshape(HBN, K, C) for ix in idx_h]
    f1b_h = [f1b[h * HBN:(h + 1) * HBN] for h in range(NH)]

    def stats0_half(h):
        return pl.pallas_call(
            _stats0_kernel,
            grid=grid,
            in_specs=[in3, in2],
            out_specs=[vec, vec],
            out_shape=[vec_shape, vec_shape],
        )(t2g_h[h], f1b_h[h])

    def _accum(parts, i):
        tot = parts[0][i]
        for p in parts[1:]:
            tot = tot + p[i]
        return tot

    st0 = [stats0_half(h) for h in range(NH)]
    s0, ss0 = _accum(st0, 0), _accum(st0, 1)

    def _affine(s, ss, gamma, beta):
        mean = s / R
        var = ss / R - mean * mean
        scale = gamma.reshape(1, C) / jnp.sqrt(var + _EPS)
        shift = beta.reshape(1, C) - mean * scale
        return scale, shift

    sc0, sh0 = _affine(s0, ss0, gamma0, beta0)
    w1t = jnp.transpose(W1).astype(jnp.bfloat16)
    b1r = b1.reshape(1, C)

    def layer1_half(h):
        return pl.pallas_call(
            functools.partial(_layer1_kernel, gt=GT, k=K, c=C),
            grid=grid,
            in_specs=[in3, in2, vec, vec,
                      pl.BlockSpec((C, C), lambda i: (0, 0)), vec],
            out_specs=[in3, vec, vec],
            out_shape=[jax.ShapeDtypeStruct((HBN, K, C), jnp.bfloat16),
                       vec_shape, vec_shape],
        )(t2g_h[h], f1b_h[h], sc0, sh0, w1t, b1r)

    l1 = [layer1_half(h) for h in range(NH)]
    sc1, sh1 = _affine(_accum(l1, 1), _accum(l1, 2), gamma1, beta1)
    w2t = jnp.transpose(W2).astype(jnp.bfloat16)
    b2r = b2.reshape(1, C)

    def layer2_half(h):
        return pl.pallas_call(
            functools.partial(_layer2_kernel, gt=GT, k=K, c=C),
            grid=grid,
            in_specs=[in3, vec, vec,
                      pl.BlockSpec((C, C), lambda i: (0, 0)), vec],
            out_specs=[in2, in2, vec, vec],
            out_shape=[jax.ShapeDtypeStruct((HBN, C), f32),
                       jax.ShapeDtypeStruct((HBN, C), f32),
                       vec_shape, vec_shape],
        )(l1[h][0], sc1, sh1, w2t, b2r)

    l2 = [layer2_half(h) for h in range(NH)]
    sc2, sh2 = _affine(_accum(l2, 2), _accum(l2, 3), gamma2, beta2)

    def final_half(h):
        return pl.pallas_call(
            _final_kernel,
            grid=grid,
            in_specs=[in2, in2, vec, vec],
            out_specs=in2,
            out_shape=jax.ShapeDtypeStruct((HBN, C), f32),
        )(l2[h][0], l2[h][1], sc2, sh2)

    outr = jnp.concatenate([final_half(h) for h in range(NH)], axis=0)
    return jnp.transpose(outr.reshape(B, N, C), (0, 2, 1))
